# Initial kernel scaffold; baseline (speedup 1.0000x reference)
import jax
import jax.numpy as jnp
from jax.experimental import pallas as pl


def kernel(x, embedding_weight):
    raise NotImplementedError("write your pallas kernel here")


# speedup vs baseline: 2.0450x; 2.0450x over previous
import functools

import numpy as np
import jax
import jax.numpy as jnp
from jax import lax
from jax.experimental import pallas as pl
from jax.experimental.pallas import tpu as pltpu
from jax.experimental.pallas import tpu_sc as plsc

_NC, _NS = 2, 16
_NW = _NC * _NS
_CHUNK = 128
_LANES = 16


def _pos_encoding_np(max_seq_len, embed_dim, n=10000.0):
    position = np.arange(max_seq_len, dtype=np.float32)[:, None]
    division_term = np.exp(
        np.arange(0, embed_dim, 2, dtype=np.float32) * (-np.log(n) / embed_dim))
    pe = np.zeros((max_seq_len, embed_dim), dtype=np.float32)
    pe[:, 0::2] = np.sin(position * division_term)
    pe[:, 1::2] = np.cos(position * division_term)
    return pe


@functools.partial(jax.jit, static_argnames=("seq_len",))
def _sc_embed(table, idx3d, pe, *, seq_len):
    n_chunks = idx3d.shape[1]
    rows_per_w = n_chunks * _CHUNK
    total_rows = _NW * rows_per_w
    d = table.shape[1]
    n_vregs = d // _LANES

    mesh = plsc.VectorSubcoreMesh(core_axis_name="c", subcore_axis_name="s",
                                  num_cores=_NC, num_subcores=_NS)

    @functools.partial(
        pl.kernel,
        out_type=jax.ShapeDtypeStruct((total_rows, d), jnp.float32),
        mesh=mesh,
        scratch_types=[
            pltpu.VMEM((n_chunks, _CHUNK), jnp.int32),
            pltpu.VMEM((seq_len, d), jnp.float32),
            pltpu.VMEM((_CHUNK, d), jnp.float32),
            pltpu.SemaphoreType.DMA,
        ],
    )
    def body(table_hbm, idx_hbm, pe_hbm, out_hbm, idx_v, pe_v, rows_v, gsem):
        w = lax.axis_index("s") * _NC + lax.axis_index("c")
        pltpu.sync_copy(idx_hbm.at[w], idx_v)
        pltpu.sync_copy(pe_hbm, pe_v)
        row0 = w * rows_per_w

        def chunk_body(k, carry):
            pltpu.async_copy(table_hbm.at[idx_v.at[k]], rows_v, gsem).wait()
            base = row0 + k * _CHUNK

            def row_body(i, carry2):
                p = lax.rem(base + i, seq_len)
                for j in range(n_vregs):
                    sl = pl.ds(j * _LANES, _LANES)
                    rows_v[i, sl] = rows_v[i, sl] + pe_v[p, sl]
                return carry2

            lax.fori_loop(0, _CHUNK, row_body, 0, unroll=False)
            pltpu.sync_copy(rows_v, out_hbm.at[pl.ds(base, _CHUNK)])
            return carry

        lax.fori_loop(0, n_chunks, chunk_body, 0, unroll=False)

    return body(table, idx3d, pe)


def kernel(x, embedding_weight):
    b, s = x.shape
    d = embedding_weight.shape[1]
    total = b * s
    assert total % (_NW * _CHUNK) == 0
    n_chunks = total // (_NW * _CHUNK)
    idx3d = x.astype(jnp.int32).reshape(_NW, n_chunks, _CHUNK)
    pe = jnp.asarray(_pos_encoding_np(s, d))
    out = _sc_embed(embedding_weight, idx3d, pe, seq_len=s)
    return out.reshape(b, s, d)


# speedup vs baseline: 3.0567x; 1.4947x over previous
import functools

import numpy as np
import jax
import jax.numpy as jnp
from jax import lax
from jax.experimental import pallas as pl
from jax.experimental.pallas import tpu as pltpu
from jax.experimental.pallas import tpu_sc as plsc

_NC, _NS = 2, 16
_NW = _NC * _NS
_HALF = 100


def _pos_encoding_np(max_seq_len, embed_dim, n=10000.0):
    position = np.arange(max_seq_len, dtype=np.float32)[:, None]
    division_term = np.exp(
        np.arange(0, embed_dim, 2, dtype=np.float32) * (-np.log(n) / embed_dim))
    pe = np.zeros((max_seq_len, embed_dim), dtype=np.float32)
    pe[:, 0::2] = np.sin(position * division_term)
    pe[:, 1::2] = np.cos(position * division_term)
    return pe


@functools.partial(jax.jit, static_argnames=("seq_len",))
def _sc_embed(table, idx3d, pe, *, seq_len):
    n_half = idx3d.shape[1]
    rows_per_w = n_half * _HALF
    n_chunks = rows_per_w // seq_len
    total_rows = _NW * rows_per_w
    d = table.shape[1]

    mesh = plsc.VectorSubcoreMesh(core_axis_name="c", subcore_axis_name="s",
                                  num_cores=_NC, num_subcores=_NS)

    @functools.partial(
        pl.kernel,
        out_type=jax.ShapeDtypeStruct((total_rows, d), jnp.float32),
        mesh=mesh,
        scratch_types=[
            pltpu.VMEM((n_half, _HALF), jnp.int32),
            pltpu.VMEM((seq_len, d), jnp.float32),
            pltpu.VMEM((seq_len, d), jnp.float32),
            pltpu.SemaphoreType.DMA,
            pltpu.SemaphoreType.DMA,
            pltpu.SemaphoreType.DMA,
            pltpu.SemaphoreType.DMA,
            pltpu.SemaphoreType.DMA,
        ],
    )
    def body(table_hbm, idx_hbm, pe_hbm, out_hbm,
             idx_v, buf0, buf1, gsem0, gsem1, osem0, osem1, psem):
        w = lax.axis_index("s") * _NC + lax.axis_index("c")
        pltpu.sync_copy(idx_hbm.at[w], idx_v)
        row0 = w * rows_per_w

        bufs = (buf0, buf1)
        gsems = (gsem0, gsem1)
        osems = (osem0, osem1)
        halves_per_chunk = seq_len // _HALF

        def start_gather(k):
            nb = k % 2
            pltpu.async_copy(pe_hbm, bufs[nb], psem).wait()
            descs = []
            for h in range(halves_per_chunk):
                descs.append(pltpu.async_copy(
                    table_hbm.at[idx_v.at[k * halves_per_chunk + h]],
                    bufs[nb].at[pl.ds(h * _HALF, _HALF)],
                    gsems[nb], add=True))
            return descs

        pending_scatter = [None, None]
        gather_descs = start_gather(0)
        for k in range(n_chunks):
            cur = k % 2
            nxt = (k + 1) % 2
            if k + 1 < n_chunks:
                if pending_scatter[nxt] is not None:
                    pending_scatter[nxt].wait()
                    pending_scatter[nxt] = None
                next_descs = start_gather(k + 1)
            else:
                next_descs = None
            for dsc in gather_descs:
                dsc.wait()
            pending_scatter[cur] = pltpu.async_copy(
                bufs[cur], out_hbm.at[pl.ds(row0 + k * seq_len, seq_len)],
                osems[cur])
            gather_descs = next_descs
        for ps in pending_scatter:
            if ps is not None:
                ps.wait()

    return body(table, idx3d, pe)


def kernel(x, embedding_weight):
    b, s = x.shape
    d = embedding_weight.shape[1]
    total = b * s
    assert total % (_NW * _HALF) == 0 and s % _HALF == 0
    n_half = total // (_NW * _HALF)
    idx3d = x.astype(jnp.int32).reshape(_NW, n_half, _HALF)
    pe = jnp.asarray(_pos_encoding_np(s, d))
    out = _sc_embed(embedding_weight, idx3d, pe, seq_len=s)
    return out.reshape(b, s, d)


# speedup vs baseline: 7.1473x; 2.3382x over previous
import functools

import numpy as np
import jax
import jax.numpy as jnp
from jax import lax
from jax.experimental import pallas as pl
from jax.experimental.pallas import tpu as pltpu
from jax.experimental.pallas import tpu_sc as plsc

_NC, _NS = 2, 16
_NW = _NC * _NS
_HALF = 100


def _pos_encoding_np(max_seq_len, embed_dim, n=10000.0):
    position = np.arange(max_seq_len, dtype=np.float32)[:, None]
    division_term = np.exp(
        np.arange(0, embed_dim, 2, dtype=np.float32) * (-np.log(n) / embed_dim))
    pe = np.zeros((max_seq_len, embed_dim), dtype=np.float32)
    pe[:, 0::2] = np.sin(position * division_term)
    pe[:, 1::2] = np.cos(position * division_term)
    return pe


@functools.partial(jax.jit, static_argnames=("seq_len",))
def _sc_embed(table, idx3d, pe, *, seq_len):
    n_half = idx3d.shape[1]
    rows_per_w = n_half * _HALF
    n_chunks = rows_per_w // seq_len
    total_rows = _NW * rows_per_w
    d = table.shape[1]

    mesh = plsc.VectorSubcoreMesh(core_axis_name="c", subcore_axis_name="s",
                                  num_cores=_NC, num_subcores=_NS)

    @functools.partial(
        pl.kernel,
        out_type=jax.ShapeDtypeStruct((total_rows, d), jnp.float32),
        mesh=mesh,
        scratch_types=[
            pltpu.VMEM((n_half, _HALF), jnp.int32),
            pltpu.VMEM((seq_len, d), jnp.float32),
            pltpu.VMEM((seq_len, d), jnp.float32),
            pltpu.SemaphoreType.DMA,
            pltpu.SemaphoreType.DMA,
            pltpu.SemaphoreType.DMA,
            pltpu.SemaphoreType.DMA,
            pltpu.SemaphoreType.DMA,
            pltpu.VMEM_SHARED((seq_len, d), jnp.float32),
        ],
    )
    def body(table_hbm, idx_hbm, pe_hbm, out_hbm,
             idx_v, buf0, buf1, gsem0, gsem1, osem0, osem1, psem, pe_sh):
        sid = lax.axis_index("s")
        w = sid * _NC + lax.axis_index("c")
        pltpu.sync_copy(idx_hbm.at[w], idx_v)

        @pl.when(sid == 0)
        def _stage_pe():
            pltpu.sync_copy(pe_hbm, buf0)
            pltpu.sync_copy(buf0, pe_sh)

        plsc.subcore_barrier()
        row0 = w * rows_per_w

        bufs = (buf0, buf1)
        gsems = (gsem0, gsem1)
        osems = (osem0, osem1)
        halves_per_chunk = seq_len // _HALF

        def start_gather(k):
            nb = k % 2
            pltpu.async_copy(pe_sh, bufs[nb], psem).wait()
            descs = []
            for h in range(halves_per_chunk):
                descs.append(pltpu.async_copy(
                    table_hbm.at[idx_v.at[k * halves_per_chunk + h]],
                    bufs[nb].at[pl.ds(h * _HALF, _HALF)],
                    gsems[nb], add=True))
            return descs

        pending_scatter = [None, None]
        gather_descs = start_gather(0)
        for k in range(n_chunks):
            cur = k % 2
            nxt = (k + 1) % 2
            if k + 1 < n_chunks:
                if pending_scatter[nxt] is not None:
                    pending_scatter[nxt].wait()
                    pending_scatter[nxt] = None
                next_descs = start_gather(k + 1)
            else:
                next_descs = None
            for dsc in gather_descs:
                dsc.wait()
            pending_scatter[cur] = pltpu.async_copy(
                bufs[cur], out_hbm.at[pl.ds(row0 + k * seq_len, seq_len)],
                osems[cur])
            gather_descs = next_descs
        for ps in pending_scatter:
            if ps is not None:
                ps.wait()

    return body(table, idx3d, pe)


def kernel(x, embedding_weight):
    b, s = x.shape
    d = embedding_weight.shape[1]
    total = b * s
    assert total % (_NW * _HALF) == 0 and s % _HALF == 0
    n_half = total // (_NW * _HALF)
    idx3d = x.astype(jnp.int32).reshape(_NW, n_half, _HALF)
    pe = jnp.asarray(_pos_encoding_np(s, d))
    out = _sc_embed(embedding_weight, idx3d, pe, seq_len=s)
    return out.reshape(b, s, d)


# speedup vs baseline: 7.3408x; 1.0271x over previous
import functools

import numpy as np
import jax
import jax.numpy as jnp
from jax import lax
from jax.experimental import pallas as pl
from jax.experimental.pallas import tpu as pltpu
from jax.experimental.pallas import tpu_sc as plsc

_NC, _NS = 2, 16
_NW = _NC * _NS
_HALF = 100
_NB = 4


def _pos_encoding_np(max_seq_len, embed_dim, n=10000.0):
    position = np.arange(max_seq_len, dtype=np.float32)[:, None]
    division_term = np.exp(
        np.arange(0, embed_dim, 2, dtype=np.float32) * (-np.log(n) / embed_dim))
    pe = np.zeros((max_seq_len, embed_dim), dtype=np.float32)
    pe[:, 0::2] = np.sin(position * division_term)
    pe[:, 1::2] = np.cos(position * division_term)
    return pe


@functools.partial(jax.jit, static_argnames=("seq_len",))
def _sc_embed(table, idx3d, pe, *, seq_len):
    n_half = idx3d.shape[1]
    rows_per_w = n_half * _HALF
    n_chunks = rows_per_w // seq_len
    total_rows = _NW * rows_per_w
    d = table.shape[1]
    halves_per_chunk = seq_len // _HALF

    mesh = plsc.VectorSubcoreMesh(core_axis_name="c", subcore_axis_name="s",
                                  num_cores=_NC, num_subcores=_NS)

    scratch = (
        [pltpu.VMEM((n_half, _HALF), jnp.int32)]
        + [pltpu.VMEM((seq_len, d), jnp.float32) for _ in range(_NB)]
        + [pltpu.SemaphoreType.DMA for _ in range(3 * _NB)]
        + [pltpu.VMEM_SHARED((seq_len, d), jnp.float32)]
    )

    @functools.partial(
        pl.kernel,
        out_type=jax.ShapeDtypeStruct((total_rows, d), jnp.float32),
        mesh=mesh,
        scratch_types=scratch,
    )
    def body(table_hbm, idx_hbm, pe_hbm, out_hbm, idx_v, *rest):
        bufs = rest[:_NB]
        gsems = rest[_NB:2 * _NB]
        osems = rest[2 * _NB:3 * _NB]
        psems = rest[3 * _NB:4 * _NB]
        pe_sh = rest[4 * _NB]

        sid = lax.axis_index("s")
        w = sid * _NC + lax.axis_index("c")
        pltpu.sync_copy(idx_hbm.at[w], idx_v)

        @pl.when(sid == 0)
        def _stage_pe():
            pltpu.sync_copy(pe_hbm, bufs[0])
            pltpu.sync_copy(bufs[0], pe_sh)

        plsc.subcore_barrier()
        row0 = w * rows_per_w

        def start_pe_init(b):
            return pltpu.async_copy(pe_sh, bufs[b], psems[b])

        def start_gather(k):
            nb = k % _NB
            return [
                pltpu.async_copy(
                    table_hbm.at[idx_v.at[k * halves_per_chunk + h]],
                    bufs[nb].at[pl.ds(h * _HALF, _HALF)],
                    gsems[nb], add=True)
                for h in range(halves_per_chunk)
            ]

        pe_descs = [None] * _NB
        scat = [None] * n_chunks
        pe_descs[0] = start_pe_init(0)
        if n_chunks > 1:
            pe_descs[1 % _NB] = start_pe_init(1 % _NB)
        pe_descs[0].wait()
        gath = start_gather(0)

        for k in range(n_chunks):
            cur = k % _NB
            if k + 2 < n_chunks:
                j = k + 2 - _NB
                if j >= 0:
                    scat[j].wait()
                    scat[j] = None
                pe_descs[(k + 2) % _NB] = start_pe_init((k + 2) % _NB)
            if k + 1 < n_chunks:
                pe_descs[(k + 1) % _NB].wait()
                next_gath = start_gather(k + 1)
            else:
                next_gath = None
            for dsc in gath:
                dsc.wait()
            scat[k] = pltpu.async_copy(
                bufs[cur], out_hbm.at[pl.ds(row0 + k * seq_len, seq_len)],
                osems[cur])
            gath = next_gath

        for j in range(n_chunks):
            if scat[j] is not None:
                scat[j].wait()

    return body(table, idx3d, pe)


def kernel(x, embedding_weight):
    b, s = x.shape
    d = embedding_weight.shape[1]
    total = b * s
    assert total % (_NW * _HALF) == 0 and s % _HALF == 0
    n_half = total // (_NW * _HALF)
    idx3d = x.astype(jnp.int32).reshape(_NW, n_half, _HALF)
    pe = jnp.asarray(_pos_encoding_np(s, d))
    out = _sc_embed(embedding_weight, idx3d, pe, seq_len=s)
    return out.reshape(b, s, d)
